# Initial kernel scaffold; baseline (speedup 1.0000x reference)
import jax
import jax.numpy as jnp
from jax.experimental import pallas as pl


def kernel(x, tables):
    raise NotImplementedError("write your pallas kernel here")


# speedup vs baseline: 3.2424x; 3.2424x over previous
import functools

import jax
import jax.numpy as jnp
from jax import lax
from jax.experimental import pallas as pl
from jax.experimental.pallas import tpu as pltpu
from jax.experimental.pallas import tpu_sc as plsc

NUM_FIELDS = 26
VOCAB = 100000
EMBED_DIM = 32
BATCH = 4096
SEQ = 20

NC = 2
NS = 16
NW = NC * NS

TOTAL = BATCH * SEQ * NUM_FIELDS
PER_W = TOTAL // NW
PERIOD = 208
CHUNK = 8 * PERIOD
NCHUNK = PER_W // CHUNK
VECS = CHUNK // 16

assert PER_W % CHUNK == 0 and CHUNK % 8 == 0


def _make_gather():
    mesh = plsc.VectorSubcoreMesh(core_axis_name="c", subcore_axis_name="s")

    @functools.partial(
        pl.kernel,
        mesh=mesh,
        out_type=jax.ShapeDtypeStruct((TOTAL, EMBED_DIM), jnp.float32),
        scratch_types=[
            pltpu.VMEM((CHUNK,), jnp.int32),
            pltpu.VMEM((CHUNK, EMBED_DIM), jnp.float32),
            pltpu.VMEM((PERIOD,), jnp.int32),
            pltpu.SemaphoreType.DMA,
        ],
        compiler_params=pltpu.CompilerParams(use_tc_tiling_on_sc=False),
    )
    def gather_kernel(tab_hbm, x_hbm, off_hbm, out_hbm, idx_v, rows_v, off_v, sem):
        wid = lax.axis_index("s") * NC + lax.axis_index("c")
        base = wid * PER_W
        pltpu.sync_copy(off_hbm, off_v)

        def chunk_body(c, carry):
            cbase = base + c * CHUNK
            pltpu.sync_copy(x_hbm.at[pl.ds(cbase, CHUNK)], idx_v)
            for j in range(VECS):
                sl = pl.ds(j * 16, 16)
                idx_v[sl] = idx_v[sl] + off_v[pl.ds((j % 13) * 16, 16)]
            pltpu.async_copy(tab_hbm.at[idx_v], rows_v, sem).wait()
            pltpu.sync_copy(rows_v, out_hbm.at[pl.ds(cbase, CHUNK)])
            return carry

        lax.fori_loop(0, NCHUNK, chunk_body, 0)

    return gather_kernel


_gather = _make_gather()


def kernel(x, tables):
    tables_flat = tables.reshape(NUM_FIELDS * VOCAB, EMBED_DIM)
    x_flat = x.reshape(-1).astype(jnp.int32)
    offs = (jnp.arange(PERIOD, dtype=jnp.int32) % NUM_FIELDS) * VOCAB
    out_flat = _gather(tables_flat, x_flat, offs)
    return out_flat.reshape(BATCH, SEQ, NUM_FIELDS * EMBED_DIM)


# speedup vs baseline: 3.3260x; 1.0258x over previous
import functools

import jax
import jax.numpy as jnp
from jax import lax
from jax.experimental import pallas as pl
from jax.experimental.pallas import tpu as pltpu
from jax.experimental.pallas import tpu_sc as plsc

NUM_FIELDS = 26
VOCAB = 100000
EMBED_DIM = 32
BATCH = 4096
SEQ = 20

NC = 2
NS = 16
NW = NC * NS

TOTAL = BATCH * SEQ * NUM_FIELDS
PER_W = TOTAL // NW
PERIOD = 208
CHUNK = 8 * PERIOD
NCHUNK = PER_W // CHUNK
VECS = CHUNK // 16

assert PER_W % CHUNK == 0 and CHUNK % 8 == 0 and NCHUNK % 2 == 0


def _make_gather():
    mesh = plsc.VectorSubcoreMesh(core_axis_name="c", subcore_axis_name="s")

    @functools.partial(
        pl.kernel,
        mesh=mesh,
        out_type=jax.ShapeDtypeStruct((TOTAL, EMBED_DIM), jnp.float32),
        scratch_types=[
            pltpu.VMEM((CHUNK,), jnp.int32),
            pltpu.VMEM((CHUNK,), jnp.int32),
            pltpu.VMEM((CHUNK, EMBED_DIM), jnp.float32),
            pltpu.VMEM((CHUNK, EMBED_DIM), jnp.float32),
            pltpu.VMEM((PERIOD,), jnp.int32),
            pltpu.SemaphoreType.DMA,
            pltpu.SemaphoreType.DMA,
            pltpu.SemaphoreType.DMA,
            pltpu.SemaphoreType.DMA,
            pltpu.SemaphoreType.DMA,
            pltpu.SemaphoreType.DMA,
        ],
        compiler_params=pltpu.CompilerParams(use_tc_tiling_on_sc=False),
    )
    def gather_kernel(tab_hbm, x_hbm, off_hbm, out_hbm,
                      idx0, idx1, rows0, rows1, off_v,
                      si0, si1, sg0, sg1, sw0, sw1):
        wid = lax.axis_index("s") * NC + lax.axis_index("c")
        base = wid * PER_W
        pltpu.sync_copy(off_hbm, off_v)

        idx_b = (idx0, idx1)
        rows_b = (rows0, rows1)
        si = (si0, si1)
        sg = (sg0, sg1)
        sw = (sw0, sw1)

        def idx_src(c):
            return x_hbm.at[pl.ds(base + c * CHUNK, CHUNK)]

        def out_dst(c):
            return out_hbm.at[pl.ds(base + c * CHUNK, CHUNK)]

        pltpu.async_copy(idx_src(0), idx0, si0)
        pltpu.async_copy(idx_src(1), idx1, si1)

        def process(c, b, first):
            pltpu.make_async_copy(idx_src(c), idx_b[b], si[b]).wait()
            for j in range(VECS):
                sl = pl.ds(j * 16, 16)
                idx_b[b][sl] = idx_b[b][sl] + off_v[pl.ds((j % 13) * 16, 16)]
            if not first:
                pltpu.make_async_copy(rows_b[b], out_dst(c), sw[b]).wait()
            pltpu.async_copy(tab_hbm.at[idx_b[b]], rows_b[b], sg[b])
            pltpu.make_async_copy(tab_hbm.at[idx_b[b]], rows_b[b], sg[b]).wait()
            pltpu.async_copy(rows_b[b], out_dst(c), sw[b])
            if isinstance(c, int):
                if c + 2 < NCHUNK:
                    pltpu.async_copy(idx_src(c + 2), idx_b[b], si[b])
            else:
                @pl.when(c + 2 < NCHUNK)
                def _():
                    pltpu.async_copy(idx_src(c + 2), idx_b[b], si[b])

        process(0, 0, True)
        process(1, 1, True)

        def loop_body(i, carry):
            process(2 * i, 0, False)
            process(2 * i + 1, 1, False)
            return carry

        lax.fori_loop(1, NCHUNK // 2, loop_body, 0)

        pltpu.make_async_copy(rows0, out_dst(NCHUNK - 2), sw0).wait()
        pltpu.make_async_copy(rows1, out_dst(NCHUNK - 1), sw1).wait()

    return gather_kernel


_gather = _make_gather()


def kernel(x, tables):
    tables_flat = tables.reshape(NUM_FIELDS * VOCAB, EMBED_DIM)
    x_flat = x.reshape(-1).astype(jnp.int32)
    offs = (jnp.arange(PERIOD, dtype=jnp.int32) % NUM_FIELDS) * VOCAB
    out_flat = _gather(tables_flat, x_flat, offs)
    return out_flat.reshape(BATCH, SEQ, NUM_FIELDS * EMBED_DIM)


# speedup vs baseline: 4.1880x; 1.2592x over previous
import functools

import jax
import jax.numpy as jnp
from jax import lax
from jax.experimental import pallas as pl
from jax.experimental.pallas import tpu as pltpu
from jax.experimental.pallas import tpu_sc as plsc

NUM_FIELDS = 26
VOCAB = 100000
EMBED_DIM = 32
BATCH = 4096
SEQ = 20

NC = 2
NS = 16
NW = NC * NS


def _make_lookup():
    mesh = plsc.VectorSubcoreMesh(core_axis_name="c", subcore_axis_name="s")

    @functools.partial(
        pl.kernel,
        mesh=mesh,
        out_type=jax.ShapeDtypeStruct((SEQ, NUM_FIELDS * EMBED_DIM, BATCH),
                                      jnp.float32),
        scratch_types=[
            pltpu.VMEM((VOCAB,), jnp.float32),
            pltpu.VMEM((BATCH,), jnp.int32),
            pltpu.VMEM((BATCH,), jnp.float32),
        ],
        compiler_params=pltpu.CompilerParams(needs_layout_passes=False),
    )
    def lookup_kernel(tabT_hbm, xT_hbm, out_hbm, row_v, idx_v, out_v):
        d = lax.axis_index("s") * NC + lax.axis_index("c")

        def f_body(f, carry):
            pltpu.sync_copy(tabT_hbm.at[f, d], row_v)

            def s_body(s, carry2):
                pltpu.sync_copy(xT_hbm.at[f, s], idx_v)

                def g_body(j, carry3):
                    sl = pl.ds(j * 16, 16)
                    out_v[sl] = plsc.load_gather(row_v, [idx_v[sl]])
                    return carry3

                lax.fori_loop(0, BATCH // 16, g_body, 0)
                pltpu.sync_copy(out_v, out_hbm.at[s, f * EMBED_DIM + d])
                return carry2

            lax.fori_loop(0, SEQ, s_body, 0)
            return carry

        lax.fori_loop(0, NUM_FIELDS, f_body, 0)

    return lookup_kernel


_lookup = _make_lookup()


def kernel(x, tables):
    tabT = jnp.transpose(tables, (0, 2, 1))
    xT = jnp.transpose(x, (2, 1, 0))
    out3 = _lookup(tabT, xT)
    return jnp.transpose(out3, (2, 0, 1))


# speedup vs baseline: 8.6629x; 2.0685x over previous
import functools

import jax
import jax.numpy as jnp
from jax import lax
from jax.experimental import pallas as pl
from jax.experimental.pallas import tpu as pltpu
from jax.experimental.pallas import tpu_sc as plsc

NUM_FIELDS = 26
VOCAB = 100000
EMBED_DIM = 32
BATCH = 4096
SEQ = 20

NC = 2
NS = 16
NW = NC * NS


def _make_lookup():
    mesh = plsc.VectorSubcoreMesh(core_axis_name="c", subcore_axis_name="s")

    @functools.partial(
        pl.kernel,
        mesh=mesh,
        out_type=jax.ShapeDtypeStruct((SEQ, NUM_FIELDS * EMBED_DIM, BATCH),
                                      jnp.float32),
        scratch_types=[
            pltpu.VMEM((VOCAB,), jnp.float32),
            pltpu.VMEM((BATCH,), jnp.int32),
            pltpu.VMEM((BATCH,), jnp.int32),
            pltpu.VMEM((BATCH,), jnp.float32),
            pltpu.VMEM((BATCH,), jnp.float32),
            pltpu.SemaphoreType.DMA,
            pltpu.SemaphoreType.DMA,
            pltpu.SemaphoreType.DMA,
            pltpu.SemaphoreType.DMA,
        ],
        compiler_params=pltpu.CompilerParams(needs_layout_passes=False),
    )
    def lookup_kernel(tabT_hbm, xT_hbm, out_hbm, row_v,
                      idx0, idx1, out0, out1, si0, si1, sw0, sw1):
        d = lax.axis_index("s") * NC + lax.axis_index("c")
        idx_b = (idx0, idx1)
        out_b = (out0, out1)
        si = (si0, si1)
        sw = (sw0, sw1)

        pltpu.async_copy(xT_hbm.at[0, 0], idx0, si0)
        pltpu.async_copy(xT_hbm.at[0, 1], idx1, si1)

        def step(f, s, b, first):
            pltpu.make_async_copy(xT_hbm.at[f, s], idx_b[b], si[b]).wait()
            if not first:
                pltpu.make_async_copy(
                    out_b[b], out_hbm.at[s, f * EMBED_DIM + d], sw[b]).wait()

            def g_body(j, carry):
                for u in range(16):
                    sl = pl.ds(j * 256 + u * 16, 16)
                    out_b[b][sl] = plsc.load_gather(row_v, [idx_b[b][sl]])
                return carry

            lax.fori_loop(0, BATCH // 256, g_body, 0)
            pltpu.async_copy(out_b[b], out_hbm.at[s, f * EMBED_DIM + d], sw[b])

            if isinstance(s, int):
                nf, ns = (f, s + 2) if s + 2 < SEQ else (f + 1, s + 2 - SEQ)
                if nf < NUM_FIELDS:
                    pltpu.async_copy(xT_hbm.at[nf, ns], idx_b[b], si[b])
            else:
                wrap = s + 2 >= SEQ
                nf = jnp.where(wrap, f + 1, f)
                ns = jnp.where(wrap, s + 2 - SEQ, s + 2)

                @pl.when(nf < NUM_FIELDS)
                def _():
                    pltpu.async_copy(xT_hbm.at[nf, ns], idx_b[b], si[b])

        pltpu.sync_copy(tabT_hbm.at[0, d], row_v)
        step(0, 0, 0, True)
        step(0, 1, 1, True)

        def spair_body(q, f):
            step(f, 2 * q, 0, False)
            step(f, 2 * q + 1, 1, False)
            return f

        lax.fori_loop(1, SEQ // 2, spair_body, 0)

        def f_body(f, carry):
            pltpu.sync_copy(tabT_hbm.at[f, d], row_v)
            lax.fori_loop(0, SEQ // 2, spair_body, f)
            return carry

        lax.fori_loop(1, NUM_FIELDS, f_body, 0)

        pltpu.make_async_copy(out0, out_hbm.at[0, d], sw0).wait()
        pltpu.make_async_copy(out1, out_hbm.at[0, d], sw1).wait()

    return lookup_kernel


_lookup = _make_lookup()


def kernel(x, tables):
    tabT = jnp.transpose(tables, (0, 2, 1))
    xT = jnp.transpose(x, (2, 1, 0))
    out3 = _lookup(tabT, xT)
    return jnp.transpose(out3, (2, 0, 1))


# speedup vs baseline: 8.7634x; 1.0116x over previous
import functools

import jax
import jax.numpy as jnp
from jax import lax
from jax.experimental import pallas as pl
from jax.experimental.pallas import tpu as pltpu
from jax.experimental.pallas import tpu_sc as plsc

NUM_FIELDS = 26
VOCAB = 100000
EMBED_DIM = 32
BATCH = 4096
SEQ = 20

NC = 2
NS = 16
NW = NC * NS


def _make_lookup():
    mesh = plsc.VectorSubcoreMesh(core_axis_name="c", subcore_axis_name="s")

    @functools.partial(
        pl.kernel,
        mesh=mesh,
        out_type=jax.ShapeDtypeStruct((SEQ, NUM_FIELDS * EMBED_DIM, BATCH),
                                      jnp.float32),
        scratch_types=[
            pltpu.VMEM((VOCAB,), jnp.float32),
            pltpu.VMEM((BATCH,), jnp.int32),
            pltpu.VMEM((BATCH,), jnp.int32),
            pltpu.VMEM((BATCH,), jnp.float32),
            pltpu.VMEM((BATCH,), jnp.float32),
            pltpu.VMEM_SHARED((2, SEQ, BATCH), jnp.int32),
            pltpu.SemaphoreType.DMA,
            pltpu.SemaphoreType.DMA,
            pltpu.SemaphoreType.DMA,
            pltpu.SemaphoreType.DMA,
            pltpu.SemaphoreType.DMA,
            pltpu.SemaphoreType.DMA,
        ],
        compiler_params=pltpu.CompilerParams(needs_layout_passes=False),
    )
    def lookup_kernel(tabT_hbm, xT_hbm, out_hbm, row_v,
                      idx0, idx1, out0, out1, xsh,
                      si0, si1, sw0, sw1, sxh, sr):
        cid = lax.axis_index("c")
        sid = lax.axis_index("s")
        d = sid * NC + cid
        idx_b = (idx0, idx1)
        out_b = (out0, out1)
        si = (si0, si1)
        sw = (sw0, sw1)

        def step(f, slot, s, b, first):
            pltpu.make_async_copy(xsh.at[slot, s], idx_b[b], si[b]).wait()
            if not first:
                pltpu.make_async_copy(
                    out_b[b], out_hbm.at[s, f * EMBED_DIM + d], sw[b]).wait()

            def g_body(j, carry):
                for u in range(16):
                    sl = pl.ds(j * 256 + u * 16, 16)
                    out_b[b][sl] = plsc.load_gather(row_v, [idx_b[b][sl]])
                return carry

            lax.fori_loop(0, BATCH // 256, g_body, 0)
            pltpu.async_copy(out_b[b], out_hbm.at[s, f * EMBED_DIM + d], sw[b])

            if isinstance(s, int):
                if s + 2 < SEQ:
                    pltpu.async_copy(xsh.at[slot, s + 2], idx_b[b], si[b])
            else:
                @pl.when(s + 2 < SEQ)
                def _():
                    pltpu.async_copy(xsh.at[slot, s + 2], idx_b[b], si[b])

        def field(f, slot, first):
            pltpu.async_copy(tabT_hbm.at[f, d], row_v, sr)

            @pl.when(sid == 0)
            def _():
                pltpu.make_async_copy(xT_hbm.at[f], xsh.at[slot], sxh).wait()

            plsc.subcore_barrier()

            @pl.when(jnp.logical_and(sid == 0, f + 1 < NUM_FIELDS))
            def _():
                pltpu.async_copy(xT_hbm.at[f + 1], xsh.at[(f + 1) % 2], sxh)

            pltpu.async_copy(xsh.at[slot, 0], idx0, si0)
            pltpu.async_copy(xsh.at[slot, 1], idx1, si1)
            pltpu.make_async_copy(tabT_hbm.at[f, d], row_v, sr).wait()

            if first:
                step(f, slot, 0, 0, True)
                step(f, slot, 1, 1, True)
            else:
                step(f, slot, 0, 0, False)
                step(f, slot, 1, 1, False)

            def spair_body(q, carry):
                step(f, slot, 2 * q, 0, False)
                step(f, slot, 2 * q + 1, 1, False)
                return carry

            lax.fori_loop(1, SEQ // 2, spair_body, 0)

        @pl.when(sid == 0)
        def _():
            pltpu.async_copy(xT_hbm.at[0], xsh.at[0], sxh)

        field(0, 0, True)

        def f_body(f, carry):
            field(f, f % 2, False)
            return carry

        lax.fori_loop(1, NUM_FIELDS, f_body, 0)

        pltpu.make_async_copy(out0, out_hbm.at[0, d], sw0).wait()
        pltpu.make_async_copy(out1, out_hbm.at[0, d], sw1).wait()

    return lookup_kernel


_lookup = _make_lookup()


def kernel(x, tables):
    tabT = jnp.transpose(tables, (0, 2, 1))
    xT = jnp.transpose(x, (2, 1, 0))
    out3 = _lookup(tabT, xT)
    return jnp.transpose(out3, (2, 0, 1))


# speedup vs baseline: 8.7743x; 1.0012x over previous
import functools

import jax
import jax.numpy as jnp
from jax import lax
from jax.experimental import pallas as pl
from jax.experimental.pallas import tpu as pltpu
from jax.experimental.pallas import tpu_sc as plsc

NUM_FIELDS = 26
VOCAB = 100000
EMBED_DIM = 32
BATCH = 4096
SEQ = 20

NC = 2
NS = 16
NW = NC * NS


def _make_lookup():
    mesh = plsc.VectorSubcoreMesh(core_axis_name="c", subcore_axis_name="s")

    @functools.partial(
        pl.kernel,
        mesh=mesh,
        out_type=jax.ShapeDtypeStruct((SEQ, NUM_FIELDS * EMBED_DIM, BATCH),
                                      jnp.float32),
        scratch_types=[
            pltpu.VMEM((VOCAB,), jnp.float32),
            pltpu.VMEM((BATCH,), jnp.int32),
            pltpu.VMEM((BATCH,), jnp.int32),
            pltpu.VMEM((BATCH,), jnp.float32),
            pltpu.VMEM((BATCH,), jnp.float32),
            pltpu.VMEM_SHARED((2, 24, BATCH), jnp.int32),
            pltpu.SemaphoreType.DMA,
            pltpu.SemaphoreType.DMA,
            pltpu.SemaphoreType.DMA,
            pltpu.SemaphoreType.DMA,
            pltpu.SemaphoreType.DMA,
            pltpu.SemaphoreType.DMA,
        ],
        compiler_params=pltpu.CompilerParams(needs_layout_passes=False),
    )
    def lookup_kernel(tabT_hbm, xT_hbm, out_hbm, row_v,
                      idx0, idx1, out0, out1, xsh,
                      si0, si1, sw0, sw1, sxh, sr):
        cid = lax.axis_index("c")
        sid = lax.axis_index("s")
        d = sid * NC + cid
        idx_b = (idx0, idx1)
        out_b = (out0, out1)
        si = (si0, si1)
        sw = (sw0, sw1)

        def step(f, slot, s, b, first):
            pltpu.make_async_copy(xsh.at[slot, s], idx_b[b], si[b]).wait()
            if not first:
                pltpu.make_async_copy(
                    out_b[b], out_hbm.at[s, f * EMBED_DIM + d], sw[b]).wait()

            def g_body(j, carry):
                for u in range(16):
                    sl = pl.ds(j * 256 + u * 16, 16)
                    out_b[b][sl] = plsc.load_gather(row_v, [idx_b[b][sl]])
                return carry

            lax.fori_loop(0, BATCH // 256, g_body, 0)
            pltpu.async_copy(out_b[b], out_hbm.at[s, f * EMBED_DIM + d], sw[b])

            if isinstance(s, int):
                if s + 2 < SEQ:
                    pltpu.async_copy(xsh.at[slot, s + 2], idx_b[b], si[b])
            else:
                @pl.when(s + 2 < SEQ)
                def _():
                    pltpu.async_copy(xsh.at[slot, s + 2], idx_b[b], si[b])

        def field(f, slot, first):
            pltpu.async_copy(tabT_hbm.at[f, d], row_v, sr)

            @pl.when(sid == 0)
            def _():
                for s in range(SEQ):
                    pltpu.make_async_copy(
                        xT_hbm.at[f, s], xsh.at[slot, s], sxh).wait()

            plsc.subcore_barrier()

            @pl.when(jnp.logical_and(sid == 0, f + 1 < NUM_FIELDS))
            def _():
                for s in range(SEQ):
                    pltpu.async_copy(
                        xT_hbm.at[f + 1, s], xsh.at[(f + 1) % 2, s], sxh)

            pltpu.async_copy(xsh.at[slot, 0], idx0, si0)
            pltpu.async_copy(xsh.at[slot, 1], idx1, si1)
            pltpu.make_async_copy(tabT_hbm.at[f, d], row_v, sr).wait()

            if first:
                step(f, slot, 0, 0, True)
                step(f, slot, 1, 1, True)
            else:
                step(f, slot, 0, 0, False)
                step(f, slot, 1, 1, False)

            def spair_body(q, carry):
                step(f, slot, 2 * q, 0, False)
                step(f, slot, 2 * q + 1, 1, False)
                return carry

            lax.fori_loop(1, SEQ // 2, spair_body, 0)

        @pl.when(sid == 0)
        def _():
            for s in range(SEQ):
                pltpu.async_copy(xT_hbm.at[0, s], xsh.at[0, s], sxh)

        field(0, 0, True)

        def f_body(f, carry):
            field(f, f % 2, False)
            return carry

        lax.fori_loop(1, NUM_FIELDS, f_body, 0)

        pltpu.make_async_copy(out0, out_hbm.at[0, d], sw0).wait()
        pltpu.make_async_copy(out1, out_hbm.at[0, d], sw1).wait()

    return lookup_kernel


_lookup = _make_lookup()


def kernel(x, tables):
    tabT = jnp.transpose(tables, (0, 2, 1))
    xT = jnp.transpose(x, (2, 1, 0))
    out3 = _lookup(tabT, xT)
    return jnp.transpose(out3, (2, 0, 1))
